# baseline (speedup 1.0000x reference)
import functools

import jax
import jax.numpy as jnp
from jax import lax
from jax.experimental import pallas as pl
from jax.experimental.pallas import tpu as pltpu
from jax.experimental.pallas import tpu_sc as plsc

_VOCAB = 64
_EMBED = 256
_NC = 2
_NS = 16
_NW = _NC * _NS
_CHUNK = 64
_NBUF = 4
_LOOKAHEAD = 2


def _sc_embed(table, idx_flat):
    B = idx_flat.shape[0]
    b_per_w = B // _NW
    n_chunks = b_per_w // _CHUNK
    mesh = plsc.VectorSubcoreMesh(core_axis_name="c", subcore_axis_name="s")

    @functools.partial(
        pl.kernel,
        mesh=mesh,
        out_type=jax.ShapeDtypeStruct((B, _EMBED), jnp.float32),
        scratch_types=(
            [pltpu.VMEM((b_per_w,), jnp.int32)]
            + [pltpu.VMEM((_CHUNK, _EMBED), jnp.float32) for _ in range(_NBUF)]
            + [pltpu.SemaphoreType.DMA for _ in range(2 * _NBUF)]
        ),
    )
    def k(table_hbm, idx_hbm, out_hbm, idx_v, *bufs_and_sems):
        rows = bufs_and_sems[:_NBUF]
        gsem = bufs_and_sems[_NBUF:2 * _NBUF]
        wsem = bufs_and_sems[2 * _NBUF:]
        wid = lax.axis_index("s") * _NC + lax.axis_index("c")
        base = wid * b_per_w

        pltpu.sync_copy(idx_hbm.at[pl.ds(base, b_per_w)], idx_v)

        def gather(j, b):
            pltpu.async_copy(
                table_hbm.at[idx_v.at[pl.ds(j * _CHUNK, _CHUNK)]],
                rows[b], gsem[b],
            )

        for j in range(_LOOKAHEAD):
            gather(j, j % _NBUF)

        def group(gidx, carry):
            i0 = gidx * _NBUF
            for b in range(_NBUF):
                i = i0 + b
                pltpu.make_async_copy(
                    table_hbm.at[idx_v.at[pl.ds(0, _CHUNK)]], rows[b], gsem[b]
                ).wait()
                pltpu.async_copy(
                    rows[b], out_hbm.at[pl.ds(base + i * _CHUNK, _CHUNK)],
                    wsem[b],
                )
                j = i + _LOOKAHEAD
                bj = (b + _LOOKAHEAD) % _NBUF

                @pl.when(jnp.logical_and(j >= _NBUF, j < n_chunks))
                def _():
                    pltpu.make_async_copy(
                        rows[bj], out_hbm.at[pl.ds(base, _CHUNK)], wsem[bj]
                    ).wait()
                    gather(j, bj)

                @pl.when(jnp.logical_and(j >= _LOOKAHEAD, j < _NBUF))
                def _():
                    gather(j, bj)
            return carry

        lax.fori_loop(0, n_chunks // _NBUF, group, 0)

        for b in range(_NBUF):
            pltpu.make_async_copy(
                rows[b], out_hbm.at[pl.ds(base, _CHUNK)], wsem[b]
            ).wait()

    return k(table, idx_flat)


def kernel(smile_input, embed_weight):
    idx = smile_input.reshape(-1).astype(jnp.int32)
    out = _sc_embed(embed_weight, idx)
    return out.reshape(smile_input.shape + (_EMBED,))


# speedup vs baseline: 6.6772x; 6.6772x over previous
import functools

import jax
import jax.numpy as jnp
from jax import lax
from jax.experimental import pallas as pl
from jax.experimental.pallas import tpu as pltpu
from jax.experimental.pallas import tpu_sc as plsc

_VOCAB = 64
_EMBED = 256
_NC = 2
_NS = 16
_NW = _NC * _NS
_G = 16
_LAG = 16


def _sc_embed(table, idx_flat):
    B = idx_flat.shape[0]
    b_per_w = B // _NW
    n_groups = b_per_w // _G
    mesh = plsc.VectorSubcoreMesh(core_axis_name="c", subcore_axis_name="s")

    @functools.partial(
        pl.kernel,
        mesh=mesh,
        out_type=jax.ShapeDtypeStruct((B, _EMBED), jnp.float32),
        scratch_types=[
            pltpu.VMEM((b_per_w,), jnp.int32),
            pltpu.VMEM((_VOCAB, _EMBED), jnp.float32),
            pltpu.SemaphoreType.DMA,
        ],
    )
    def k(table_hbm, idx_hbm, out_hbm, idx_v, table_v, sem):
        wid = lax.axis_index("s") * _NC + lax.axis_index("c")
        base = wid * b_per_w
        pltpu.sync_copy(table_hbm, table_v)
        pltpu.sync_copy(idx_hbm.at[pl.ds(base, b_per_w)], idx_v)

        def drain_one_group():
            pltpu.make_async_copy(
                table_v.at[pl.ds(0, _G)], out_hbm.at[pl.ds(base, _G)], sem
            ).wait()

        def body(g, carry):
            gvec = idx_v[pl.ds(g * _G, _G)]
            for l in range(_G):
                ridx = gvec[l]
                pltpu.async_copy(
                    table_v.at[pl.ds(ridx, 1)],
                    out_hbm.at[pl.ds(base + g * _G + l, 1)],
                    sem,
                )

            @pl.when(g >= _LAG)
            def _():
                drain_one_group()

            return carry

        lax.fori_loop(0, n_groups, body, 0)

        for _ in range(_LAG):
            drain_one_group()

    return k(table, idx_flat)


def kernel(smile_input, embed_weight):
    idx = smile_input.reshape(-1).astype(jnp.int32)
    out = _sc_embed(embed_weight, idx)
    return out.reshape(smile_input.shape + (_EMBED,))
